# baseline (speedup 1.0000x reference)
import functools

import jax
import jax.numpy as jnp
from jax import lax
from jax.experimental import pallas as pl
from jax.experimental.pallas import tpu as pltpu
from jax.experimental.pallas import tpu_sc as plsc

B, S, D = 2, 2048, 768
E, K, H = 64, 2, 64
OUT = 768
N = B * S
NK = N * K
C = 3 * (NK // E)
EC = E * C
T = 512
NT = N // T
FBLK = 128
FNB = C // FBLK

NC, NS = 2, 16
NW = NC * NS
TPW = N // NW


def _gate_body(x_ref, gw1_ref, gw2_ref,
               d1_ref, d2_ref, g1_ref, g2_ref, s1_ref, s2_ref, loss_ref,
               nblk_ref, run_ref, imp_ref, load_ref):
    i = pl.program_id(0)

    @pl.when(i == 0)
    def _():
        run_ref[...] = jnp.zeros((1, E), jnp.float32)
        imp_ref[...] = jnp.zeros((1, E), jnp.float32)
        load_ref[...] = jnp.zeros((1, E), jnp.float32)

    xb = x_ref[...]
    f1 = jnp.tanh(lax.dot_general(xb, gw1_ref[...], (((1,), (1,)), ((), ())),
                                  preferred_element_type=jnp.float32))
    logits = lax.dot_general(f1, gw2_ref[...], (((1,), (1,)), ((), ())),
                             preferred_element_type=jnp.float32)

    eidx = lax.broadcasted_iota(jnp.int32, (T, E), 1)
    m1 = jnp.max(logits, axis=1, keepdims=True)
    i1 = jnp.min(jnp.where(logits == m1, eidx, E), axis=1, keepdims=True)
    masked = jnp.where(eidx == i1, -jnp.inf, logits)
    m2 = jnp.max(masked, axis=1, keepdims=True)
    i2 = jnp.min(jnp.where(masked == m2, eidx, E), axis=1, keepdims=True)

    es = jnp.exp(m2 - m1)
    s1 = 1.0 / (1.0 + es)
    s2 = es * s1

    oh1 = (eidx == i1).astype(jnp.float32)
    oh2 = (eidx == i2).astype(jnp.float32)
    ohs = oh1 + oh2

    imp_ref[...] += jnp.sum(oh1 * s1 + oh2 * s2, axis=0, keepdims=True)
    load_tile = jnp.sum(ohs, axis=0, keepdims=True)
    load_ref[...] += load_tile

    rr = lax.broadcasted_iota(jnp.int32, (T, T), 0)
    cc = lax.broadcasted_iota(jnp.int32, (T, T), 1)
    tri = (cc < rr).astype(jnp.float32)
    pref = lax.dot_general(tri, ohs, (((1,), (0,)), ((), ())),
                           preferred_element_type=jnp.float32,
                           precision=lax.Precision.HIGHEST)
    tot = run_ref[...] + pref
    rank1 = jnp.sum(tot * oh1, axis=1, keepdims=True)
    rank2 = jnp.sum(tot * oh2, axis=1, keepdims=True)
    run_ref[...] += load_tile

    cap = jnp.float32(C)
    keep1 = rank1 < cap
    keep2 = rank2 < cap
    s1_ref[...] = jnp.broadcast_to(jnp.where(keep1, s1, 0.0), (T, 16))
    s2_ref[...] = jnp.broadcast_to(jnp.where(keep2, s2, 0.0), (T, 16))
    slot1 = jnp.minimum(rank1, cap - 1.0).astype(jnp.int32)
    slot2 = jnp.minimum(rank2, cap - 1.0).astype(jnp.int32)
    comb1 = i1 * C + slot1
    comb2 = i2 * C + slot2
    g1_ref[...] = comb1
    g2_ref[...] = comb2
    d1_ref[...] = jnp.where(keep1, comb1, EC)
    d2_ref[...] = jnp.where(keep2, comb2, EC)

    @pl.when(i == NT - 1)
    def _():
        def cv2(v):
            mean = jnp.sum(v) / E
            var = jnp.sum((v - mean) * (v - mean)) / (E - 1)
            return var / (mean * mean + 1e-10)
        loss = (cv2(imp_ref[...]) + cv2(load_ref[...])) * 0.01
        loss_ref[...] = jnp.full((8, 128), loss, jnp.float32)
        cnt = jnp.minimum(run_ref[...], jnp.float32(C))
        nb = jnp.clip(jnp.ceil(cnt * (1.0 / FBLK)), 1.0, C // FBLK)
        pad = jnp.zeros((1, 128 - E), jnp.float32)
        nblk_ref[...] = jnp.broadcast_to(
            jnp.concatenate([nb, pad], axis=1), (8, 128)).astype(jnp.int32)


_gate_call = pl.pallas_call(
    _gate_body,
    grid=(NT,),
    in_specs=[
        pl.BlockSpec((T, D), lambda i: (i, 0)),
        pl.BlockSpec((E, D), lambda i: (0, 0)),
        pl.BlockSpec((E, E), lambda i: (0, 0)),
    ],
    out_specs=[
        pl.BlockSpec((T, 1), lambda i: (i, 0)),
        pl.BlockSpec((T, 1), lambda i: (i, 0)),
        pl.BlockSpec((T, 1), lambda i: (i, 0)),
        pl.BlockSpec((T, 1), lambda i: (i, 0)),
        pl.BlockSpec((T, 16), lambda i: (i, 0)),
        pl.BlockSpec((T, 16), lambda i: (i, 0)),
        pl.BlockSpec((8, 128), lambda i: (0, 0)),
        pl.BlockSpec((8, 128), lambda i: (0, 0)),
    ],
    out_shape=[
        jax.ShapeDtypeStruct((N, 1), jnp.int32),
        jax.ShapeDtypeStruct((N, 1), jnp.int32),
        jax.ShapeDtypeStruct((N, 1), jnp.int32),
        jax.ShapeDtypeStruct((N, 1), jnp.int32),
        jax.ShapeDtypeStruct((N, 16), jnp.float32),
        jax.ShapeDtypeStruct((N, 16), jnp.float32),
        jax.ShapeDtypeStruct((8, 128), jnp.float32),
        jax.ShapeDtypeStruct((8, 128), jnp.int32),
    ],
    scratch_shapes=[
        pltpu.VMEM((1, E), jnp.float32),
        pltpu.VMEM((1, E), jnp.float32),
        pltpu.VMEM((1, E), jnp.float32),
    ],
)


def _dispatch_body(x_hbm, d1_hbm, d2_hbm, xd_hbm, xbuf, d1v, d2v, sem):
    wid = lax.axis_index("s") * NC + lax.axis_index("c")
    base = wid * TPW
    pltpu.sync_copy(d1_hbm.at[pl.ds(base, TPW)], d1v)
    pltpu.sync_copy(d2_hbm.at[pl.ds(base, TPW)], d2v)
    pltpu.sync_copy(x_hbm.at[pl.ds(base, TPW)], xbuf)
    pltpu.async_copy(xbuf, xd_hbm.at[d1v], sem).wait()
    pltpu.async_copy(xbuf, xd_hbm.at[d2v], sem).wait()


@functools.cache
def _dispatch_call():
    return pl.kernel(
        _dispatch_body,
        out_type=jax.ShapeDtypeStruct((EC + 8, D), jnp.float32),
        mesh=plsc.VectorSubcoreMesh(core_axis_name="c", subcore_axis_name="s",
                                    num_cores=NC, num_subcores=NS),
        scratch_types=[
            pltpu.VMEM((TPW, D), jnp.float32),
            pltpu.VMEM((TPW,), jnp.int32),
            pltpu.VMEM((TPW,), jnp.int32),
            pltpu.SemaphoreType.DMA,
        ],
    )


def _ffn_body(nblk_ref, xd_ref, wu_ref, bu_ref, wd_ref, bd_ref, yd_ref):
    e = pl.program_id(0)
    b = pl.program_id(1)

    @pl.when(b < nblk_ref[e])
    def _():
        xb = xd_ref[...]
        up = lax.dot_general(xb, wu_ref[0], (((1,), (1,)), ((), ())),
                             preferred_element_type=jnp.float32) + bu_ref[0]
        h = up * (1.0 / (1.0 + jnp.exp(-up)))
        dn = lax.dot_general(h, wd_ref[0], (((1,), (1,)), ((), ())),
                             preferred_element_type=jnp.float32) + bd_ref[0]
        yd_ref[...] = dn


def _blk_map(e, b, nblk_ref):
    return (e * FNB + jnp.minimum(b, nblk_ref[e] - 1), 0)


_ffn_call = pl.pallas_call(
    _ffn_body,
    grid_spec=pltpu.PrefetchScalarGridSpec(
        num_scalar_prefetch=1,
        grid=(E, FNB),
        in_specs=[
            pl.BlockSpec((FBLK, D), _blk_map),
            pl.BlockSpec((1, H, D), lambda e, b, n: (e, 0, 0)),
            pl.BlockSpec((1, 1, H), lambda e, b, n: (e, 0, 0)),
            pl.BlockSpec((1, OUT, H), lambda e, b, n: (e, 0, 0)),
            pl.BlockSpec((1, 1, OUT), lambda e, b, n: (e, 0, 0)),
        ],
        out_specs=pl.BlockSpec((FBLK, OUT), _blk_map),
    ),
    out_shape=jax.ShapeDtypeStruct((EC, OUT), jnp.float32),
)


_CCH = 32
_VPR = OUT // 16


def _combine_body(yd_hbm, d1_hbm, d2_hbm, s1_hbm, s2_hbm, y_hbm,
                  buf1, buf2, ybuf, d1v, d2v, s1v, s2v, sem1, sem2):
    wid = lax.axis_index("s") * NC + lax.axis_index("c")
    base = wid * TPW
    for ch in range(TPW // _CCH):
        off = base + ch * _CCH
        pltpu.sync_copy(d1_hbm.at[pl.ds(off, _CCH)], d1v)
        pltpu.sync_copy(d2_hbm.at[pl.ds(off, _CCH)], d2v)
        pltpu.sync_copy(s1_hbm.at[pl.ds(off, _CCH)], s1v)
        pltpu.sync_copy(s2_hbm.at[pl.ds(off, _CCH)], s2v)
        h1 = pltpu.async_copy(yd_hbm.at[d1v], buf1, sem1)
        h2 = pltpu.async_copy(yd_hbm.at[d2v], buf2, sem2)
        h1.wait()
        h2.wait()

        def body(t, carry):
            sv1 = s1v[t, pl.ds(0, 16)]
            sv2 = s2v[t, pl.ds(0, 16)]
            for c in range(_VPR):
                sl = pl.ds(c * 16, 16)
                ybuf[t, sl] = sv1 * buf1[t, sl] + sv2 * buf2[t, sl]
            return carry

        lax.fori_loop(0, _CCH, body, 0)
        pltpu.sync_copy(ybuf, y_hbm.at[pl.ds(off, _CCH)])


@functools.cache
def _combine_call():
    return pl.kernel(
        _combine_body,
        out_type=jax.ShapeDtypeStruct((N, OUT), jnp.float32),
        mesh=plsc.VectorSubcoreMesh(core_axis_name="c", subcore_axis_name="s",
                                    num_cores=NC, num_subcores=NS),
        scratch_types=[
            pltpu.VMEM((_CCH, OUT), jnp.float32),
            pltpu.VMEM((_CCH, OUT), jnp.float32),
            pltpu.VMEM((_CCH, OUT), jnp.float32),
            pltpu.VMEM((_CCH,), jnp.int32),
            pltpu.VMEM((_CCH,), jnp.int32),
            pltpu.VMEM((_CCH, 16), jnp.float32),
            pltpu.VMEM((_CCH, 16), jnp.float32),
            pltpu.SemaphoreType.DMA,
            pltpu.SemaphoreType.DMA,
        ],
    )


@jax.jit
def kernel(x, gate_w1, gate_w2, w_up, b_up, w_down, b_down):
    orig_shape = x.shape
    xf = x.reshape(-1, D)

    d1c, d2c, g1c, g2c, s1c, s2c, loss_arr, nblk_arr = _gate_call(
        xf, gate_w1, gate_w2)

    xd = _dispatch_call()(xf, d1c[:, 0], d2c[:, 0])
    yd = _ffn_call(nblk_arr[0, :E], xd, w_up, b_up.reshape(E, 1, H), w_down,
                   b_down.reshape(E, 1, OUT))
    y = _combine_call()(yd, g1c[:, 0], g2c[:, 0], s1c, s2c)

    return y.reshape(orig_shape[:-1] + (OUT,)), loss_arr[0, 0]


# speedup vs baseline: 1.3931x; 1.3931x over previous
import functools

import jax
import jax.numpy as jnp
from jax import lax
from jax.experimental import pallas as pl
from jax.experimental.pallas import tpu as pltpu
from jax.experimental.pallas import tpu_sc as plsc

B, S, D = 2, 2048, 768
E, K, H = 64, 2, 64
OUT = 768
N = B * S
NK = N * K
C = 3 * (NK // E)
EC = E * C
T = 512
NT = N // T
FBLK = 128
FNB = C // FBLK

NC, NS = 2, 16
NW = NC * NS
TPW = N // NW


def _gate_body(x_ref, gw1_ref, gw2_ref,
               d1_ref, d2_ref, g1_ref, g2_ref, s1_ref, s2_ref, loss_ref,
               nblk_ref, xbf_ref, run_ref, imp_ref, load_ref):
    i = pl.program_id(0)

    @pl.when(i == 0)
    def _():
        run_ref[...] = jnp.zeros((1, E), jnp.float32)
        imp_ref[...] = jnp.zeros((1, E), jnp.float32)
        load_ref[...] = jnp.zeros((1, E), jnp.float32)

    xb = x_ref[...]
    xb16 = lax.bitcast_convert_type(xb.astype(jnp.bfloat16), jnp.uint16)
    lo32 = xb16[:, :D // 2].astype(jnp.uint32)
    hi32 = xb16[:, D // 2:].astype(jnp.uint32)
    xbf_ref[...] = lax.bitcast_convert_type((hi32 << 16) | lo32, jnp.int32)
    f1 = jnp.tanh(lax.dot_general(xb, gw1_ref[...], (((1,), (1,)), ((), ())),
                                  preferred_element_type=jnp.float32))
    logits = lax.dot_general(f1, gw2_ref[...], (((1,), (1,)), ((), ())),
                             preferred_element_type=jnp.float32)

    eidx = lax.broadcasted_iota(jnp.int32, (T, E), 1)
    m1 = jnp.max(logits, axis=1, keepdims=True)
    i1 = jnp.min(jnp.where(logits == m1, eidx, E), axis=1, keepdims=True)
    masked = jnp.where(eidx == i1, -jnp.inf, logits)
    m2 = jnp.max(masked, axis=1, keepdims=True)
    i2 = jnp.min(jnp.where(masked == m2, eidx, E), axis=1, keepdims=True)

    es = jnp.exp(m2 - m1)
    s1 = 1.0 / (1.0 + es)
    s2 = es * s1

    oh1 = (eidx == i1).astype(jnp.float32)
    oh2 = (eidx == i2).astype(jnp.float32)
    ohs = oh1 + oh2

    imp_ref[...] += jnp.sum(oh1 * s1 + oh2 * s2, axis=0, keepdims=True)
    load_tile = jnp.sum(ohs, axis=0, keepdims=True)
    load_ref[...] += load_tile

    rr = lax.broadcasted_iota(jnp.int32, (T, T), 0)
    cc = lax.broadcasted_iota(jnp.int32, (T, T), 1)
    tri = (cc < rr).astype(jnp.float32)
    pref = lax.dot_general(tri, ohs, (((1,), (0,)), ((), ())),
                           preferred_element_type=jnp.float32,
                           precision=lax.Precision.HIGHEST)
    tot = run_ref[...] + pref
    rank1 = jnp.sum(tot * oh1, axis=1, keepdims=True)
    rank2 = jnp.sum(tot * oh2, axis=1, keepdims=True)
    run_ref[...] += load_tile

    cap = jnp.float32(C)
    keep1 = rank1 < cap
    keep2 = rank2 < cap
    s1_ref[...] = jnp.broadcast_to(jnp.where(keep1, s1, 0.0), (T, 16))
    s2_ref[...] = jnp.broadcast_to(jnp.where(keep2, s2, 0.0), (T, 16))
    slot1 = jnp.minimum(rank1, cap - 1.0).astype(jnp.int32)
    slot2 = jnp.minimum(rank2, cap - 1.0).astype(jnp.int32)
    comb1 = i1 * C + slot1
    comb2 = i2 * C + slot2
    g1_ref[...] = comb1
    g2_ref[...] = comb2
    d1_ref[...] = jnp.where(keep1, comb1, EC)
    d2_ref[...] = jnp.where(keep2, comb2, EC)

    @pl.when(i == NT - 1)
    def _():
        def cv2(v):
            mean = jnp.sum(v) / E
            var = jnp.sum((v - mean) * (v - mean)) / (E - 1)
            return var / (mean * mean + 1e-10)
        loss = (cv2(imp_ref[...]) + cv2(load_ref[...])) * 0.01
        loss_ref[...] = jnp.full((8, 128), loss, jnp.float32)
        cnt = jnp.minimum(run_ref[...], jnp.float32(C))
        nb = jnp.clip(jnp.ceil(cnt * (1.0 / FBLK)), 1.0, C // FBLK)
        pad = jnp.zeros((1, 128 - E), jnp.float32)
        nblk_ref[...] = jnp.broadcast_to(
            jnp.concatenate([nb, pad], axis=1), (8, 128)).astype(jnp.int32)


_gate_call = pl.pallas_call(
    _gate_body,
    grid=(NT,),
    in_specs=[
        pl.BlockSpec((T, D), lambda i: (i, 0)),
        pl.BlockSpec((E, D), lambda i: (0, 0)),
        pl.BlockSpec((E, E), lambda i: (0, 0)),
    ],
    out_specs=[
        pl.BlockSpec((T, 1), lambda i: (i, 0)),
        pl.BlockSpec((T, 1), lambda i: (i, 0)),
        pl.BlockSpec((T, 1), lambda i: (i, 0)),
        pl.BlockSpec((T, 1), lambda i: (i, 0)),
        pl.BlockSpec((T, 16), lambda i: (i, 0)),
        pl.BlockSpec((T, 16), lambda i: (i, 0)),
        pl.BlockSpec((8, 128), lambda i: (0, 0)),
        pl.BlockSpec((8, 128), lambda i: (0, 0)),
        pl.BlockSpec((T, D // 2), lambda i: (i, 0)),
    ],
    out_shape=[
        jax.ShapeDtypeStruct((N, 1), jnp.int32),
        jax.ShapeDtypeStruct((N, 1), jnp.int32),
        jax.ShapeDtypeStruct((N, 1), jnp.int32),
        jax.ShapeDtypeStruct((N, 1), jnp.int32),
        jax.ShapeDtypeStruct((N, 16), jnp.float32),
        jax.ShapeDtypeStruct((N, 16), jnp.float32),
        jax.ShapeDtypeStruct((8, 128), jnp.float32),
        jax.ShapeDtypeStruct((8, 128), jnp.int32),
        jax.ShapeDtypeStruct((N, D // 2), jnp.int32),
    ],
    scratch_shapes=[
        pltpu.VMEM((1, E), jnp.float32),
        pltpu.VMEM((1, E), jnp.float32),
        pltpu.VMEM((1, E), jnp.float32),
    ],
)


def _dispatch_body(x_hbm, d1_hbm, d2_hbm, xd_hbm, xbuf, d1v, d2v, sem):
    wid = lax.axis_index("s") * NC + lax.axis_index("c")
    base = wid * TPW
    pltpu.sync_copy(d1_hbm.at[pl.ds(base, TPW)], d1v)
    pltpu.sync_copy(d2_hbm.at[pl.ds(base, TPW)], d2v)
    pltpu.sync_copy(x_hbm.at[pl.ds(base, TPW)], xbuf)
    pltpu.async_copy(xbuf, xd_hbm.at[d1v], sem).wait()
    pltpu.async_copy(xbuf, xd_hbm.at[d2v], sem).wait()


@functools.cache
def _dispatch_call():
    return pl.kernel(
        _dispatch_body,
        out_type=jax.ShapeDtypeStruct((EC + 8, D // 2), jnp.int32),
        mesh=plsc.VectorSubcoreMesh(core_axis_name="c", subcore_axis_name="s",
                                    num_cores=NC, num_subcores=NS),
        scratch_types=[
            pltpu.VMEM((TPW, D // 2), jnp.int32),
            pltpu.VMEM((TPW,), jnp.int32),
            pltpu.VMEM((TPW,), jnp.int32),
            pltpu.SemaphoreType.DMA,
        ],
    )


def _ffn_body(xd_ref, wu_ref, bu_ref, wd_ref, bd_ref, yd_ref):
    w32 = lax.bitcast_convert_type(xd_ref[...], jnp.uint32)
    lo = lax.bitcast_convert_type(
        (w32 & 0xFFFF).astype(jnp.uint16), jnp.bfloat16).astype(jnp.float32)
    hi = lax.bitcast_convert_type(
        (w32 >> 16).astype(jnp.uint16), jnp.bfloat16).astype(jnp.float32)
    wu = wu_ref[0]
    up = (lax.dot_general(lo, wu[:, :D // 2], (((1,), (1,)), ((), ())),
                          preferred_element_type=jnp.float32)
          + lax.dot_general(hi, wu[:, D // 2:], (((1,), (1,)), ((), ())),
                            preferred_element_type=jnp.float32)
          + bu_ref[0])
    h = up * (1.0 / (1.0 + jnp.exp(-up)))
    dn = lax.dot_general(h, wd_ref[0], (((1,), (1,)), ((), ())),
                         preferred_element_type=jnp.float32) + bd_ref[0]
    yd_ref[...] = dn


_ffn_call = pl.pallas_call(
    _ffn_body,
    grid=(E,),
    in_specs=[
        pl.BlockSpec((C, D // 2), lambda e: (e, 0)),
        pl.BlockSpec((1, H, D), lambda e: (e, 0, 0)),
        pl.BlockSpec((1, 1, H), lambda e: (e, 0, 0)),
        pl.BlockSpec((1, OUT, H), lambda e: (e, 0, 0)),
        pl.BlockSpec((1, 1, OUT), lambda e: (e, 0, 0)),
    ],
    out_specs=pl.BlockSpec((C, OUT), lambda e: (e, 0)),
    out_shape=jax.ShapeDtypeStruct((EC, OUT), jnp.float32),
)


_CCH = 32
_VPR = OUT // 16


def _combine_body(yd_hbm, d1_hbm, d2_hbm, s1_hbm, s2_hbm, y_hbm,
                  buf1, buf2, ybuf, d1v, d2v, s1v, s2v, sem1, sem2):
    wid = lax.axis_index("s") * NC + lax.axis_index("c")
    base = wid * TPW
    for ch in range(TPW // _CCH):
        off = base + ch * _CCH
        pltpu.sync_copy(d1_hbm.at[pl.ds(off, _CCH)], d1v)
        pltpu.sync_copy(d2_hbm.at[pl.ds(off, _CCH)], d2v)
        pltpu.sync_copy(s1_hbm.at[pl.ds(off, _CCH)], s1v)
        pltpu.sync_copy(s2_hbm.at[pl.ds(off, _CCH)], s2v)
        h1 = pltpu.async_copy(yd_hbm.at[d1v], buf1, sem1)
        h2 = pltpu.async_copy(yd_hbm.at[d2v], buf2, sem2)
        h1.wait()
        h2.wait()

        def body(t, carry):
            sv1 = s1v[t, pl.ds(0, 16)]
            sv2 = s2v[t, pl.ds(0, 16)]
            for c in range(_VPR):
                sl = pl.ds(c * 16, 16)
                ybuf[t, sl] = sv1 * buf1[t, sl] + sv2 * buf2[t, sl]
            return carry

        lax.fori_loop(0, _CCH, body, 0)
        pltpu.sync_copy(ybuf, y_hbm.at[pl.ds(off, _CCH)])


@functools.cache
def _combine_call():
    return pl.kernel(
        _combine_body,
        out_type=jax.ShapeDtypeStruct((N, OUT), jnp.float32),
        mesh=plsc.VectorSubcoreMesh(core_axis_name="c", subcore_axis_name="s",
                                    num_cores=NC, num_subcores=NS),
        scratch_types=[
            pltpu.VMEM((_CCH, OUT), jnp.float32),
            pltpu.VMEM((_CCH, OUT), jnp.float32),
            pltpu.VMEM((_CCH, OUT), jnp.float32),
            pltpu.VMEM((_CCH,), jnp.int32),
            pltpu.VMEM((_CCH,), jnp.int32),
            pltpu.VMEM((_CCH, 16), jnp.float32),
            pltpu.VMEM((_CCH, 16), jnp.float32),
            pltpu.SemaphoreType.DMA,
            pltpu.SemaphoreType.DMA,
        ],
    )


@jax.jit
def kernel(x, gate_w1, gate_w2, w_up, b_up, w_down, b_down):
    orig_shape = x.shape
    xf = x.reshape(-1, D)

    d1c, d2c, g1c, g2c, s1c, s2c, loss_arr, nblk_arr, xbf = _gate_call(
        xf, gate_w1, gate_w2)

    xd = _dispatch_call()(xbf, d1c[:, 0], d2c[:, 0])
    del nblk_arr
    yd = _ffn_call(xd, w_up, b_up.reshape(E, 1, H), w_down,
                   b_down.reshape(E, 1, OUT))
    y = _combine_call()(yd, g1c[:, 0], g2c[:, 0], s1c, s2c)

    return y.reshape(orig_shape[:-1] + (OUT,)), loss_arr[0, 0]


# speedup vs baseline: 1.5478x; 1.1111x over previous
import functools

import jax
import jax.numpy as jnp
from jax import lax
from jax.experimental import pallas as pl
from jax.experimental.pallas import tpu as pltpu
from jax.experimental.pallas import tpu_sc as plsc

B, S, D = 2, 2048, 768
E, K, H = 64, 2, 64
OUT = 768
N = B * S
NK = N * K
C = 3 * (NK // E)
EC = E * C
T = 512
NT = N // T
FBLK = 128
FNB = C // FBLK

NC, NS = 2, 16
NW = NC * NS
TPW = N // NW


def _gate_body(x_ref, gw1_ref, gw2_ref,
               d1_ref, d2_ref, g1_ref, g2_ref, s1_ref, s2_ref, loss_ref,
               nblk_ref, xbf_ref, run_ref, imp_ref, load_ref):
    i = pl.program_id(0)

    @pl.when(i == 0)
    def _():
        run_ref[...] = jnp.zeros((1, E), jnp.float32)
        imp_ref[...] = jnp.zeros((1, E), jnp.float32)
        load_ref[...] = jnp.zeros((1, E), jnp.float32)

    xb = x_ref[...]
    xb16 = lax.bitcast_convert_type(xb.astype(jnp.bfloat16), jnp.uint16)
    lo32 = xb16[:, :D // 2].astype(jnp.uint32)
    hi32 = xb16[:, D // 2:].astype(jnp.uint32)
    xbf_ref[...] = lax.bitcast_convert_type((hi32 << 16) | lo32, jnp.int32)
    f1 = jnp.tanh(lax.dot_general(xb, gw1_ref[...], (((1,), (1,)), ((), ())),
                                  preferred_element_type=jnp.float32))
    logits = lax.dot_general(f1, gw2_ref[...], (((1,), (1,)), ((), ())),
                             preferred_element_type=jnp.float32)

    eidx = lax.broadcasted_iota(jnp.int32, (T, E), 1)
    m1 = jnp.max(logits, axis=1, keepdims=True)
    i1 = jnp.min(jnp.where(logits == m1, eidx, E), axis=1, keepdims=True)
    masked = jnp.where(eidx == i1, -jnp.inf, logits)
    m2 = jnp.max(masked, axis=1, keepdims=True)
    i2 = jnp.min(jnp.where(masked == m2, eidx, E), axis=1, keepdims=True)

    es = jnp.exp(m2 - m1)
    s1 = 1.0 / (1.0 + es)
    s2 = es * s1

    oh1 = (eidx == i1).astype(jnp.float32)
    oh2 = (eidx == i2).astype(jnp.float32)
    ohs = oh1 + oh2

    imp_ref[...] += jnp.sum(oh1 * s1 + oh2 * s2, axis=0, keepdims=True)
    load_tile = jnp.sum(ohs, axis=0, keepdims=True)
    load_ref[...] += load_tile

    rr = lax.broadcasted_iota(jnp.int32, (T, T), 0)
    cc = lax.broadcasted_iota(jnp.int32, (T, T), 1)
    tri = (cc < rr).astype(jnp.float32)
    pref = lax.dot_general(tri, ohs, (((1,), (0,)), ((), ())),
                           preferred_element_type=jnp.float32,
                           precision=lax.Precision.HIGHEST)
    tot = run_ref[...] + pref
    rank1 = jnp.sum(tot * oh1, axis=1, keepdims=True)
    rank2 = jnp.sum(tot * oh2, axis=1, keepdims=True)
    run_ref[...] += load_tile

    cap = jnp.float32(C)
    keep1 = rank1 < cap
    keep2 = rank2 < cap
    s1_ref[...] = jnp.broadcast_to(jnp.where(keep1, s1, 0.0), (T, 16))
    s2_ref[...] = jnp.broadcast_to(jnp.where(keep2, s2, 0.0), (T, 16))
    slot1 = jnp.minimum(rank1, cap - 1.0).astype(jnp.int32)
    slot2 = jnp.minimum(rank2, cap - 1.0).astype(jnp.int32)
    comb1 = i1 * C + slot1
    comb2 = i2 * C + slot2
    g1_ref[...] = comb1
    g2_ref[...] = comb2
    d1_ref[...] = jnp.where(keep1, comb1, EC)
    d2_ref[...] = jnp.where(keep2, comb2, EC)

    @pl.when(i == NT - 1)
    def _():
        def cv2(v):
            mean = jnp.sum(v) / E
            var = jnp.sum((v - mean) * (v - mean)) / (E - 1)
            return var / (mean * mean + 1e-10)
        loss = (cv2(imp_ref[...]) + cv2(load_ref[...])) * 0.01
        loss_ref[...] = jnp.full((8, 128), loss, jnp.float32)
        cnt = jnp.minimum(run_ref[...], jnp.float32(C))
        nb = jnp.clip(jnp.ceil(cnt * (1.0 / FBLK)), 1.0, C // FBLK)
        pad = jnp.zeros((1, 128 - E), jnp.float32)
        nblk_ref[...] = jnp.broadcast_to(
            jnp.concatenate([nb, pad], axis=1), (8, 128)).astype(jnp.int32)


_gate_call = pl.pallas_call(
    _gate_body,
    grid=(NT,),
    in_specs=[
        pl.BlockSpec((T, D), lambda i: (i, 0)),
        pl.BlockSpec((E, D), lambda i: (0, 0)),
        pl.BlockSpec((E, E), lambda i: (0, 0)),
    ],
    out_specs=[
        pl.BlockSpec((T, 1), lambda i: (i, 0)),
        pl.BlockSpec((T, 1), lambda i: (i, 0)),
        pl.BlockSpec((T, 1), lambda i: (i, 0)),
        pl.BlockSpec((T, 1), lambda i: (i, 0)),
        pl.BlockSpec((T, 16), lambda i: (i, 0)),
        pl.BlockSpec((T, 16), lambda i: (i, 0)),
        pl.BlockSpec((8, 128), lambda i: (0, 0)),
        pl.BlockSpec((8, 128), lambda i: (0, 0)),
        pl.BlockSpec((T, D // 2), lambda i: (i, 0)),
    ],
    out_shape=[
        jax.ShapeDtypeStruct((N, 1), jnp.int32),
        jax.ShapeDtypeStruct((N, 1), jnp.int32),
        jax.ShapeDtypeStruct((N, 1), jnp.int32),
        jax.ShapeDtypeStruct((N, 1), jnp.int32),
        jax.ShapeDtypeStruct((N, 16), jnp.float32),
        jax.ShapeDtypeStruct((N, 16), jnp.float32),
        jax.ShapeDtypeStruct((8, 128), jnp.float32),
        jax.ShapeDtypeStruct((8, 128), jnp.int32),
        jax.ShapeDtypeStruct((N, D // 2), jnp.int32),
    ],
    scratch_shapes=[
        pltpu.VMEM((1, E), jnp.float32),
        pltpu.VMEM((1, E), jnp.float32),
        pltpu.VMEM((1, E), jnp.float32),
    ],
)


def _dispatch_body(x_hbm, d1_hbm, d2_hbm, xd_hbm, xbuf, d1v, d2v, sem):
    wid = lax.axis_index("s") * NC + lax.axis_index("c")
    base = wid * TPW
    pltpu.sync_copy(d1_hbm.at[pl.ds(base, TPW)], d1v)
    pltpu.sync_copy(d2_hbm.at[pl.ds(base, TPW)], d2v)
    pltpu.sync_copy(x_hbm.at[pl.ds(base, TPW)], xbuf)
    pltpu.async_copy(xbuf, xd_hbm.at[d1v], sem).wait()
    pltpu.async_copy(xbuf, xd_hbm.at[d2v], sem).wait()


@functools.cache
def _dispatch_call():
    return pl.kernel(
        _dispatch_body,
        out_type=jax.ShapeDtypeStruct((EC + 8, D // 2), jnp.int32),
        mesh=plsc.VectorSubcoreMesh(core_axis_name="c", subcore_axis_name="s",
                                    num_cores=NC, num_subcores=NS),
        scratch_types=[
            pltpu.VMEM((TPW, D // 2), jnp.int32),
            pltpu.VMEM((TPW,), jnp.int32),
            pltpu.VMEM((TPW,), jnp.int32),
            pltpu.SemaphoreType.DMA,
        ],
    )


def _ffn_body(xd_ref, wu_ref, bu_ref, wd_ref, bd_ref, yd_ref):
    w32 = lax.bitcast_convert_type(xd_ref[...], jnp.uint32)
    lo = lax.bitcast_convert_type(
        (w32 & 0xFFFF).astype(jnp.uint16), jnp.bfloat16).astype(jnp.float32)
    hi = lax.bitcast_convert_type(
        (w32 >> 16).astype(jnp.uint16), jnp.bfloat16).astype(jnp.float32)
    wu = wu_ref[0]
    up = (lax.dot_general(lo, wu[:, :D // 2], (((1,), (1,)), ((), ())),
                          preferred_element_type=jnp.float32)
          + lax.dot_general(hi, wu[:, D // 2:], (((1,), (1,)), ((), ())),
                            preferred_element_type=jnp.float32)
          + bu_ref[0])
    h = up * (1.0 / (1.0 + jnp.exp(-up)))
    dn = lax.dot_general(h, wd_ref[0], (((1,), (1,)), ((), ())),
                         preferred_element_type=jnp.float32) + bd_ref[0]
    dn16 = lax.bitcast_convert_type(dn.astype(jnp.bfloat16), jnp.uint16)
    dlo = dn16[:, :OUT // 2].astype(jnp.uint32)
    dhi = dn16[:, OUT // 2:].astype(jnp.uint32)
    yd_ref[...] = lax.bitcast_convert_type((dhi << 16) | dlo, jnp.int32)


_ffn_call = pl.pallas_call(
    _ffn_body,
    grid=(E,),
    in_specs=[
        pl.BlockSpec((C, D // 2), lambda e: (e, 0)),
        pl.BlockSpec((1, H, D), lambda e: (e, 0, 0)),
        pl.BlockSpec((1, 1, H), lambda e: (e, 0, 0)),
        pl.BlockSpec((1, OUT, H), lambda e: (e, 0, 0)),
        pl.BlockSpec((1, 1, OUT), lambda e: (e, 0, 0)),
    ],
    out_specs=pl.BlockSpec((C, OUT // 2), lambda e: (e, 0)),
    out_shape=jax.ShapeDtypeStruct((EC, OUT // 2), jnp.int32),
)


_CCH = 64


def _combine_body(yd_hbm, d1_hbm, d2_hbm, y1_hbm, y2_hbm,
                  buf1, buf2, d1v, d2v, sem1, sem2):
    wid = lax.axis_index("s") * NC + lax.axis_index("c")
    base = wid * TPW
    for ch in range(TPW // _CCH):
        off = base + ch * _CCH
        pltpu.sync_copy(d1_hbm.at[pl.ds(off, _CCH)], d1v)
        pltpu.sync_copy(d2_hbm.at[pl.ds(off, _CCH)], d2v)
        h1 = pltpu.async_copy(yd_hbm.at[d1v], buf1, sem1)
        h2 = pltpu.async_copy(yd_hbm.at[d2v], buf2, sem2)
        h1.wait()
        h2.wait()
        pltpu.sync_copy(buf1, y1_hbm.at[pl.ds(off, _CCH)])
        pltpu.sync_copy(buf2, y2_hbm.at[pl.ds(off, _CCH)])


@functools.cache
def _combine_call():
    return pl.kernel(
        _combine_body,
        out_type=(jax.ShapeDtypeStruct((N, OUT // 2), jnp.int32),
                  jax.ShapeDtypeStruct((N, OUT // 2), jnp.int32)),
        mesh=plsc.VectorSubcoreMesh(core_axis_name="c", subcore_axis_name="s",
                                    num_cores=NC, num_subcores=NS),
        scratch_types=[
            pltpu.VMEM((_CCH, OUT // 2), jnp.int32),
            pltpu.VMEM((_CCH, OUT // 2), jnp.int32),
            pltpu.VMEM((_CCH,), jnp.int32),
            pltpu.VMEM((_CCH,), jnp.int32),
            pltpu.SemaphoreType.DMA,
            pltpu.SemaphoreType.DMA,
        ],
    )


def _wadd_body(y1_ref, y2_ref, s1_ref, s2_ref, y_ref):
    def dec_lo(w32):
        return lax.bitcast_convert_type(w32 << 16, jnp.float32)

    def dec_hi(w32):
        return lax.bitcast_convert_type(w32 & jnp.int32(-65536), jnp.float32)

    w1 = y1_ref[...]
    w2 = y2_ref[...]
    s1 = s1_ref[:, :1]
    s2 = s2_ref[:, :1]
    y_ref[:, :OUT // 2] = s1 * dec_lo(w1) + s2 * dec_lo(w2)
    y_ref[:, OUT // 2:] = s1 * dec_hi(w1) + s2 * dec_hi(w2)


_wadd_call = pl.pallas_call(
    _wadd_body,
    grid=(NT,),
    in_specs=[
        pl.BlockSpec((T, OUT // 2), lambda i: (i, 0)),
        pl.BlockSpec((T, OUT // 2), lambda i: (i, 0)),
        pl.BlockSpec((T, 16), lambda i: (i, 0)),
        pl.BlockSpec((T, 16), lambda i: (i, 0)),
    ],
    out_specs=pl.BlockSpec((T, OUT), lambda i: (i, 0)),
    out_shape=jax.ShapeDtypeStruct((N, OUT), jnp.float32),
)


@jax.jit
def kernel(x, gate_w1, gate_w2, w_up, b_up, w_down, b_down):
    orig_shape = x.shape
    xf = x.reshape(-1, D)

    d1c, d2c, g1c, g2c, s1c, s2c, loss_arr, nblk_arr, xbf = _gate_call(
        xf, gate_w1, gate_w2)

    xd = _dispatch_call()(xbf, d1c[:, 0], d2c[:, 0])
    del nblk_arr
    yd = _ffn_call(xd, w_up, b_up.reshape(E, 1, H), w_down,
                   b_down.reshape(E, 1, OUT))
    y1p, y2p = _combine_call()(yd, g1c[:, 0], g2c[:, 0])
    y = _wadd_call(y1p, y2p, s1c, s2c)

    return y.reshape(orig_shape[:-1] + (OUT,)), loss_arr[0, 0]


# speedup vs baseline: 1.5766x; 1.0186x over previous
import functools

import jax
import jax.numpy as jnp
from jax import lax
from jax.experimental import pallas as pl
from jax.experimental.pallas import tpu as pltpu
from jax.experimental.pallas import tpu_sc as plsc

B, S, D = 2, 2048, 768
E, K, H = 64, 2, 64
OUT = 768
N = B * S
NK = N * K
C = 3 * (NK // E)
EC = E * C
T = 512
NT = N // T
FBLK = 128
FNB = C // FBLK

NC, NS = 2, 16
NW = NC * NS
TPW = N // NW


def _gate_body(x_ref, gw1_ref, gw2_ref,
               d1_ref, d2_ref, g1_ref, g2_ref, s1_ref, s2_ref, loss_ref,
               nblk_ref, xbf_ref, run_ref, imp_ref, load_ref):
    i = pl.program_id(0)

    @pl.when(i == 0)
    def _():
        run_ref[...] = jnp.zeros((1, E), jnp.float32)
        imp_ref[...] = jnp.zeros((1, E), jnp.float32)
        load_ref[...] = jnp.zeros((1, E), jnp.float32)

    xb = x_ref[...]
    xb16 = lax.bitcast_convert_type(xb.astype(jnp.bfloat16), jnp.uint16)
    lo32 = xb16[:, :D // 2].astype(jnp.uint32)
    hi32 = xb16[:, D // 2:].astype(jnp.uint32)
    xbf_ref[...] = lax.bitcast_convert_type((hi32 << 16) | lo32, jnp.int32)
    f1 = jnp.tanh(lax.dot_general(xb, gw1_ref[...], (((1,), (1,)), ((), ())),
                                  preferred_element_type=jnp.float32))
    logits = lax.dot_general(f1, gw2_ref[...], (((1,), (1,)), ((), ())),
                             preferred_element_type=jnp.float32)

    eidx = lax.broadcasted_iota(jnp.int32, (T, E), 1)
    m1 = jnp.max(logits, axis=1, keepdims=True)
    i1 = jnp.min(jnp.where(logits == m1, eidx, E), axis=1, keepdims=True)
    masked = jnp.where(eidx == i1, -jnp.inf, logits)
    m2 = jnp.max(masked, axis=1, keepdims=True)
    i2 = jnp.min(jnp.where(masked == m2, eidx, E), axis=1, keepdims=True)

    es = jnp.exp(m2 - m1)
    s1 = 1.0 / (1.0 + es)
    s2 = es * s1

    oh1 = (eidx == i1).astype(jnp.float32)
    oh2 = (eidx == i2).astype(jnp.float32)
    ohs = oh1 + oh2

    imp_ref[...] += jnp.sum(oh1 * s1 + oh2 * s2, axis=0, keepdims=True)
    load_tile = jnp.sum(ohs, axis=0, keepdims=True)
    load_ref[...] += load_tile

    rr = lax.broadcasted_iota(jnp.int32, (T, T), 0)
    cc = lax.broadcasted_iota(jnp.int32, (T, T), 1)
    tri = (cc < rr).astype(jnp.float32)
    pref = lax.dot_general(tri.astype(jnp.bfloat16), ohs.astype(jnp.bfloat16),
                           (((1,), (0,)), ((), ())),
                           preferred_element_type=jnp.float32)
    tot = run_ref[...] + pref
    rank1 = jnp.round(jnp.sum(tot * oh1, axis=1, keepdims=True))
    rank2 = jnp.round(jnp.sum(tot * oh2, axis=1, keepdims=True))
    run_ref[...] += load_tile

    cap = jnp.float32(C)
    keep1 = rank1 < cap
    keep2 = rank2 < cap
    s1_ref[...] = jnp.broadcast_to(jnp.where(keep1, s1, 0.0), (T, 16))
    s2_ref[...] = jnp.broadcast_to(jnp.where(keep2, s2, 0.0), (T, 16))
    slot1 = jnp.minimum(rank1, cap - 1.0).astype(jnp.int32)
    slot2 = jnp.minimum(rank2, cap - 1.0).astype(jnp.int32)
    comb1 = i1 * C + slot1
    comb2 = i2 * C + slot2
    g1_ref[...] = comb1
    g2_ref[...] = comb2
    d1_ref[...] = jnp.where(keep1, comb1, EC)
    d2_ref[...] = jnp.where(keep2, comb2, EC)

    @pl.when(i == NT - 1)
    def _():
        def cv2(v):
            mean = jnp.sum(v) / E
            var = jnp.sum((v - mean) * (v - mean)) / (E - 1)
            return var / (mean * mean + 1e-10)
        loss = (cv2(imp_ref[...]) + cv2(load_ref[...])) * 0.01
        loss_ref[...] = jnp.full((8, 128), loss, jnp.float32)
        cnt = jnp.minimum(run_ref[...], jnp.float32(C))
        nb = jnp.clip(jnp.ceil(cnt * (1.0 / FBLK)), 1.0, C // FBLK)
        pad = jnp.zeros((1, 128 - E), jnp.float32)
        nblk_ref[...] = jnp.broadcast_to(
            jnp.concatenate([nb, pad], axis=1), (8, 128)).astype(jnp.int32)


_gate_call = pl.pallas_call(
    _gate_body,
    grid=(NT,),
    in_specs=[
        pl.BlockSpec((T, D), lambda i: (i, 0)),
        pl.BlockSpec((E, D), lambda i: (0, 0)),
        pl.BlockSpec((E, E), lambda i: (0, 0)),
    ],
    out_specs=[
        pl.BlockSpec((T, 1), lambda i: (i, 0)),
        pl.BlockSpec((T, 1), lambda i: (i, 0)),
        pl.BlockSpec((T, 1), lambda i: (i, 0)),
        pl.BlockSpec((T, 1), lambda i: (i, 0)),
        pl.BlockSpec((T, 16), lambda i: (i, 0)),
        pl.BlockSpec((T, 16), lambda i: (i, 0)),
        pl.BlockSpec((8, 128), lambda i: (0, 0)),
        pl.BlockSpec((8, 128), lambda i: (0, 0)),
        pl.BlockSpec((T, D // 2), lambda i: (i, 0)),
    ],
    out_shape=[
        jax.ShapeDtypeStruct((N, 1), jnp.int32),
        jax.ShapeDtypeStruct((N, 1), jnp.int32),
        jax.ShapeDtypeStruct((N, 1), jnp.int32),
        jax.ShapeDtypeStruct((N, 1), jnp.int32),
        jax.ShapeDtypeStruct((N, 16), jnp.float32),
        jax.ShapeDtypeStruct((N, 16), jnp.float32),
        jax.ShapeDtypeStruct((8, 128), jnp.float32),
        jax.ShapeDtypeStruct((8, 128), jnp.int32),
        jax.ShapeDtypeStruct((N, D // 2), jnp.int32),
    ],
    scratch_shapes=[
        pltpu.VMEM((1, E), jnp.float32),
        pltpu.VMEM((1, E), jnp.float32),
        pltpu.VMEM((1, E), jnp.float32),
    ],
)


def _dispatch_body(x_hbm, d1_hbm, d2_hbm, xd_hbm, xbuf, d1v, d2v, sem, sem2):
    wid = lax.axis_index("s") * NC + lax.axis_index("c")
    base = wid * TPW
    pltpu.sync_copy(d1_hbm.at[pl.ds(base, TPW)], d1v)
    pltpu.sync_copy(d2_hbm.at[pl.ds(base, TPW)], d2v)
    pltpu.sync_copy(x_hbm.at[pl.ds(base, TPW)], xbuf)
    h1 = pltpu.async_copy(xbuf, xd_hbm.at[d1v], sem)
    h2 = pltpu.async_copy(xbuf, xd_hbm.at[d2v], sem2)
    h1.wait()
    h2.wait()


@functools.cache
def _dispatch_call():
    return pl.kernel(
        _dispatch_body,
        out_type=jax.ShapeDtypeStruct((EC + 8, D // 2), jnp.int32),
        mesh=plsc.VectorSubcoreMesh(core_axis_name="c", subcore_axis_name="s",
                                    num_cores=NC, num_subcores=NS),
        scratch_types=[
            pltpu.VMEM((TPW, D // 2), jnp.int32),
            pltpu.VMEM((TPW,), jnp.int32),
            pltpu.VMEM((TPW,), jnp.int32),
            pltpu.SemaphoreType.DMA,
            pltpu.SemaphoreType.DMA,
        ],
    )


def _ffn_body(xd_ref, wu_ref, bu_ref, wd_ref, bd_ref, yd_ref):
    w32 = lax.bitcast_convert_type(xd_ref[...], jnp.uint32)
    lo = lax.bitcast_convert_type((w32 & 0xFFFF).astype(jnp.uint16),
                                  jnp.bfloat16)
    hi = lax.bitcast_convert_type((w32 >> 16).astype(jnp.uint16),
                                  jnp.bfloat16)
    wu = wu_ref[0].astype(jnp.bfloat16)
    up = (lax.dot_general(lo, wu[:, :D // 2], (((1,), (1,)), ((), ())),
                          preferred_element_type=jnp.float32)
          + lax.dot_general(hi, wu[:, D // 2:], (((1,), (1,)), ((), ())),
                            preferred_element_type=jnp.float32)
          + bu_ref[0])
    h = (up * (1.0 / (1.0 + jnp.exp(-up)))).astype(jnp.bfloat16)
    dn = lax.dot_general(h, wd_ref[0].astype(jnp.bfloat16),
                         (((1,), (1,)), ((), ())),
                         preferred_element_type=jnp.float32) + bd_ref[0]
    dn16 = lax.bitcast_convert_type(dn.astype(jnp.bfloat16), jnp.uint16)
    dlo = dn16[:, :OUT // 2].astype(jnp.uint32)
    dhi = dn16[:, OUT // 2:].astype(jnp.uint32)
    yd_ref[...] = lax.bitcast_convert_type((dhi << 16) | dlo, jnp.int32)


_ffn_call = pl.pallas_call(
    _ffn_body,
    grid=(E,),
    in_specs=[
        pl.BlockSpec((C, D // 2), lambda e: (e, 0)),
        pl.BlockSpec((1, H, D), lambda e: (e, 0, 0)),
        pl.BlockSpec((1, 1, H), lambda e: (e, 0, 0)),
        pl.BlockSpec((1, OUT, H), lambda e: (e, 0, 0)),
        pl.BlockSpec((1, 1, OUT), lambda e: (e, 0, 0)),
    ],
    out_specs=pl.BlockSpec((C, OUT // 2), lambda e: (e, 0)),
    out_shape=jax.ShapeDtypeStruct((EC, OUT // 2), jnp.int32),
)


_CCH = 64


def _combine_body(yd_hbm, d1_hbm, d2_hbm, y1_hbm, y2_hbm,
                  buf1, buf2, d1v, d2v, sem1, sem2):
    wid = lax.axis_index("s") * NC + lax.axis_index("c")
    base = wid * TPW
    for ch in range(TPW // _CCH):
        off = base + ch * _CCH
        pltpu.sync_copy(d1_hbm.at[pl.ds(off, _CCH)], d1v)
        pltpu.sync_copy(d2_hbm.at[pl.ds(off, _CCH)], d2v)
        h1 = pltpu.async_copy(yd_hbm.at[d1v], buf1, sem1)
        h2 = pltpu.async_copy(yd_hbm.at[d2v], buf2, sem2)
        h1.wait()
        h2.wait()
        pltpu.sync_copy(buf1, y1_hbm.at[pl.ds(off, _CCH)])
        pltpu.sync_copy(buf2, y2_hbm.at[pl.ds(off, _CCH)])


@functools.cache
def _combine_call():
    return pl.kernel(
        _combine_body,
        out_type=(jax.ShapeDtypeStruct((N, OUT // 2), jnp.int32),
                  jax.ShapeDtypeStruct((N, OUT // 2), jnp.int32)),
        mesh=plsc.VectorSubcoreMesh(core_axis_name="c", subcore_axis_name="s",
                                    num_cores=NC, num_subcores=NS),
        scratch_types=[
            pltpu.VMEM((_CCH, OUT // 2), jnp.int32),
            pltpu.VMEM((_CCH, OUT // 2), jnp.int32),
            pltpu.VMEM((_CCH,), jnp.int32),
            pltpu.VMEM((_CCH,), jnp.int32),
            pltpu.SemaphoreType.DMA,
            pltpu.SemaphoreType.DMA,
        ],
    )


def _wadd_body(y1_ref, y2_ref, s1_ref, s2_ref, y_ref):
    def dec_lo(w32):
        return lax.bitcast_convert_type(w32 << 16, jnp.float32)

    def dec_hi(w32):
        return lax.bitcast_convert_type(w32 & jnp.int32(-65536), jnp.float32)

    w1 = y1_ref[...]
    w2 = y2_ref[...]
    s1 = s1_ref[:, :1]
    s2 = s2_ref[:, :1]
    y_ref[:, :OUT // 2] = s1 * dec_lo(w1) + s2 * dec_lo(w2)
    y_ref[:, OUT // 2:] = s1 * dec_hi(w1) + s2 * dec_hi(w2)


_wadd_call = pl.pallas_call(
    _wadd_body,
    grid=(NT,),
    in_specs=[
        pl.BlockSpec((T, OUT // 2), lambda i: (i, 0)),
        pl.BlockSpec((T, OUT // 2), lambda i: (i, 0)),
        pl.BlockSpec((T, 16), lambda i: (i, 0)),
        pl.BlockSpec((T, 16), lambda i: (i, 0)),
    ],
    out_specs=pl.BlockSpec((T, OUT), lambda i: (i, 0)),
    out_shape=jax.ShapeDtypeStruct((N, OUT), jnp.float32),
)


@jax.jit
def kernel(x, gate_w1, gate_w2, w_up, b_up, w_down, b_down):
    orig_shape = x.shape
    xf = x.reshape(-1, D)

    d1c, d2c, g1c, g2c, s1c, s2c, loss_arr, nblk_arr, xbf = _gate_call(
        xf, gate_w1, gate_w2)

    xd = _dispatch_call()(xbf, d1c[:, 0], d2c[:, 0])
    del nblk_arr
    yd = _ffn_call(xd, w_up, b_up.reshape(E, 1, H), w_down,
                   b_down.reshape(E, 1, OUT))
    y1p, y2p = _combine_call()(yd, g1c[:, 0], g2c[:, 0])
    y = _wadd_call(y1p, y2p, s1c, s2c)

    return y.reshape(orig_shape[:-1] + (OUT,)), loss_arr[0, 0]


# speedup vs baseline: 1.5769x; 1.0002x over previous
import functools

import jax
import jax.numpy as jnp
from jax import lax
from jax.experimental import pallas as pl
from jax.experimental.pallas import tpu as pltpu
from jax.experimental.pallas import tpu_sc as plsc

B, S, D = 2, 2048, 768
E, K, H = 64, 2, 64
OUT = 768
N = B * S
NK = N * K
C = 3 * (NK // E)
EC = E * C
T = 512
NT = N // T
FBLK = 128
FNB = C // FBLK

NC, NS = 2, 16
NW = NC * NS
TPW = N // NW


def _gate_body(x_ref, gw1_ref, gw2_ref,
               d1_ref, d2_ref, g1_ref, g2_ref, s1_ref, s2_ref, loss_ref,
               nblk_ref, xbf_ref, run_ref, imp_ref, load_ref):
    i = pl.program_id(0)

    @pl.when(i == 0)
    def _():
        run_ref[...] = jnp.zeros((1, E), jnp.float32)
        imp_ref[...] = jnp.zeros((1, E), jnp.float32)
        load_ref[...] = jnp.zeros((1, E), jnp.float32)

    xb = x_ref[...]
    xb16 = lax.bitcast_convert_type(xb.astype(jnp.bfloat16), jnp.uint16)
    lo32 = xb16[:, :D // 2].astype(jnp.uint32)
    hi32 = xb16[:, D // 2:].astype(jnp.uint32)
    xbf_ref[...] = lax.bitcast_convert_type((hi32 << 16) | lo32, jnp.int32)
    f1 = jnp.tanh(lax.dot_general(xb, gw1_ref[...], (((1,), (1,)), ((), ())),
                                  preferred_element_type=jnp.float32))
    logits = lax.dot_general(f1, gw2_ref[...], (((1,), (1,)), ((), ())),
                             preferred_element_type=jnp.float32)

    eidx = lax.broadcasted_iota(jnp.int32, (T, E), 1)
    m1 = jnp.max(logits, axis=1, keepdims=True)
    i1 = jnp.min(jnp.where(logits == m1, eidx, E), axis=1, keepdims=True)
    masked = jnp.where(eidx == i1, -jnp.inf, logits)
    m2 = jnp.max(masked, axis=1, keepdims=True)
    i2 = jnp.min(jnp.where(masked == m2, eidx, E), axis=1, keepdims=True)

    es = jnp.exp(m2 - m1)
    s1 = 1.0 / (1.0 + es)
    s2 = es * s1

    oh1 = (eidx == i1).astype(jnp.float32)
    oh2 = (eidx == i2).astype(jnp.float32)
    ohs = oh1 + oh2

    imp_ref[...] += jnp.sum(oh1 * s1 + oh2 * s2, axis=0, keepdims=True)
    load_tile = jnp.sum(ohs, axis=0, keepdims=True)
    load_ref[...] += load_tile

    rr = lax.broadcasted_iota(jnp.int32, (T, T), 0)
    cc = lax.broadcasted_iota(jnp.int32, (T, T), 1)
    tri = (cc < rr).astype(jnp.float32)
    pref = lax.dot_general(tri.astype(jnp.bfloat16), ohs.astype(jnp.bfloat16),
                           (((1,), (0,)), ((), ())),
                           preferred_element_type=jnp.float32)
    tot = run_ref[...] + pref
    rank1 = jnp.round(jnp.sum(tot * oh1, axis=1, keepdims=True))
    rank2 = jnp.round(jnp.sum(tot * oh2, axis=1, keepdims=True))
    run_ref[...] += load_tile

    cap = jnp.float32(C)
    keep1 = rank1 < cap
    keep2 = rank2 < cap
    s1_ref[...] = jnp.broadcast_to(jnp.where(keep1, s1, 0.0), (T, 16))
    s2_ref[...] = jnp.broadcast_to(jnp.where(keep2, s2, 0.0), (T, 16))
    slot1 = jnp.minimum(rank1, cap - 1.0).astype(jnp.int32)
    slot2 = jnp.minimum(rank2, cap - 1.0).astype(jnp.int32)
    comb1 = i1 * C + slot1
    comb2 = i2 * C + slot2
    g1_ref[...] = comb1
    g2_ref[...] = comb2
    d1_ref[...] = jnp.where(keep1, comb1, EC)
    d2_ref[...] = jnp.where(keep2, comb2, EC)

    @pl.when(i == NT - 1)
    def _():
        def cv2(v):
            mean = jnp.sum(v) / E
            var = jnp.sum((v - mean) * (v - mean)) / (E - 1)
            return var / (mean * mean + 1e-10)
        loss = (cv2(imp_ref[...]) + cv2(load_ref[...])) * 0.01
        loss_ref[...] = jnp.full((8, 128), loss, jnp.float32)
        cnt = jnp.minimum(run_ref[...], jnp.float32(C))
        nb = jnp.clip(jnp.ceil(cnt * (1.0 / FBLK)), 1.0, C // FBLK)
        pad = jnp.zeros((1, 128 - E), jnp.float32)
        nblk_ref[...] = jnp.broadcast_to(
            jnp.concatenate([nb, pad], axis=1), (8, 128)).astype(jnp.int32)


_gate_call = pl.pallas_call(
    _gate_body,
    grid=(NT,),
    in_specs=[
        pl.BlockSpec((T, D), lambda i: (i, 0)),
        pl.BlockSpec((E, D), lambda i: (0, 0)),
        pl.BlockSpec((E, E), lambda i: (0, 0)),
    ],
    out_specs=[
        pl.BlockSpec((T, 1), lambda i: (i, 0)),
        pl.BlockSpec((T, 1), lambda i: (i, 0)),
        pl.BlockSpec((T, 1), lambda i: (i, 0)),
        pl.BlockSpec((T, 1), lambda i: (i, 0)),
        pl.BlockSpec((T, 16), lambda i: (i, 0)),
        pl.BlockSpec((T, 16), lambda i: (i, 0)),
        pl.BlockSpec((8, 128), lambda i: (0, 0)),
        pl.BlockSpec((8, 128), lambda i: (0, 0)),
        pl.BlockSpec((T, D // 2), lambda i: (i, 0)),
    ],
    out_shape=[
        jax.ShapeDtypeStruct((N, 1), jnp.int32),
        jax.ShapeDtypeStruct((N, 1), jnp.int32),
        jax.ShapeDtypeStruct((N, 1), jnp.int32),
        jax.ShapeDtypeStruct((N, 1), jnp.int32),
        jax.ShapeDtypeStruct((N, 16), jnp.float32),
        jax.ShapeDtypeStruct((N, 16), jnp.float32),
        jax.ShapeDtypeStruct((8, 128), jnp.float32),
        jax.ShapeDtypeStruct((8, 128), jnp.int32),
        jax.ShapeDtypeStruct((N, D // 2), jnp.int32),
    ],
    scratch_shapes=[
        pltpu.VMEM((1, E), jnp.float32),
        pltpu.VMEM((1, E), jnp.float32),
        pltpu.VMEM((1, E), jnp.float32),
    ],
)


def _dispatch_body(x_hbm, d1_hbm, d2_hbm, xd_hbm, xbuf, d1v, d2v, sem, sem2):
    wid = lax.axis_index("s") * NC + lax.axis_index("c")
    base = wid * TPW
    pltpu.sync_copy(d1_hbm.at[pl.ds(base, TPW)], d1v)
    pltpu.sync_copy(d2_hbm.at[pl.ds(base, TPW)], d2v)
    pltpu.sync_copy(x_hbm.at[pl.ds(base, TPW)], xbuf)
    h1 = pltpu.async_copy(xbuf, xd_hbm.at[d1v], sem)
    h2 = pltpu.async_copy(xbuf, xd_hbm.at[d2v], sem2)
    h1.wait()
    h2.wait()


@functools.cache
def _dispatch_call():
    return pl.kernel(
        _dispatch_body,
        out_type=jax.ShapeDtypeStruct((EC + 8, D // 2), jnp.int32),
        mesh=plsc.VectorSubcoreMesh(core_axis_name="c", subcore_axis_name="s",
                                    num_cores=NC, num_subcores=NS),
        compiler_params=pltpu.CompilerParams(use_tc_tiling_on_sc=True),
        scratch_types=[
            pltpu.VMEM((TPW, D // 2), jnp.int32),
            pltpu.VMEM((TPW,), jnp.int32),
            pltpu.VMEM((TPW,), jnp.int32),
            pltpu.SemaphoreType.DMA,
            pltpu.SemaphoreType.DMA,
        ],
    )


def _ffn_body(xd_ref, wu_ref, bu_ref, wd_ref, bd_ref, yd_ref):
    w32 = lax.bitcast_convert_type(xd_ref[...], jnp.uint32)
    lo = lax.bitcast_convert_type((w32 & 0xFFFF).astype(jnp.uint16),
                                  jnp.bfloat16)
    hi = lax.bitcast_convert_type((w32 >> 16).astype(jnp.uint16),
                                  jnp.bfloat16)
    wu = wu_ref[0].astype(jnp.bfloat16)
    up = (lax.dot_general(lo, wu[:, :D // 2], (((1,), (1,)), ((), ())),
                          preferred_element_type=jnp.float32)
          + lax.dot_general(hi, wu[:, D // 2:], (((1,), (1,)), ((), ())),
                            preferred_element_type=jnp.float32)
          + bu_ref[0])
    h = (up * (1.0 / (1.0 + jnp.exp(-up)))).astype(jnp.bfloat16)
    dn = lax.dot_general(h, wd_ref[0].astype(jnp.bfloat16),
                         (((1,), (1,)), ((), ())),
                         preferred_element_type=jnp.float32) + bd_ref[0]
    dn16 = lax.bitcast_convert_type(dn.astype(jnp.bfloat16), jnp.uint16)
    dlo = dn16[:, :OUT // 2].astype(jnp.uint32)
    dhi = dn16[:, OUT // 2:].astype(jnp.uint32)
    yd_ref[...] = lax.bitcast_convert_type((dhi << 16) | dlo, jnp.int32)


_ffn_call = pl.pallas_call(
    _ffn_body,
    grid=(E,),
    in_specs=[
        pl.BlockSpec((C, D // 2), lambda e: (e, 0)),
        pl.BlockSpec((1, H, D), lambda e: (e, 0, 0)),
        pl.BlockSpec((1, 1, H), lambda e: (e, 0, 0)),
        pl.BlockSpec((1, OUT, H), lambda e: (e, 0, 0)),
        pl.BlockSpec((1, 1, OUT), lambda e: (e, 0, 0)),
    ],
    out_specs=pl.BlockSpec((C, OUT // 2), lambda e: (e, 0)),
    out_shape=jax.ShapeDtypeStruct((EC, OUT // 2), jnp.int32),
)


_CCH = 64


def _combine_body(yd_hbm, d1_hbm, d2_hbm, y1_hbm, y2_hbm,
                  buf1, buf2, d1v, d2v, sem1, sem2):
    wid = lax.axis_index("s") * NC + lax.axis_index("c")
    base = wid * TPW
    for ch in range(TPW // _CCH):
        off = base + ch * _CCH
        pltpu.sync_copy(d1_hbm.at[pl.ds(off, _CCH)], d1v)
        pltpu.sync_copy(d2_hbm.at[pl.ds(off, _CCH)], d2v)
        h1 = pltpu.async_copy(yd_hbm.at[d1v], buf1, sem1)
        h2 = pltpu.async_copy(yd_hbm.at[d2v], buf2, sem2)
        h1.wait()
        h2.wait()
        pltpu.sync_copy(buf1, y1_hbm.at[pl.ds(off, _CCH)])
        pltpu.sync_copy(buf2, y2_hbm.at[pl.ds(off, _CCH)])


@functools.cache
def _combine_call():
    return pl.kernel(
        _combine_body,
        out_type=(jax.ShapeDtypeStruct((N, OUT // 2), jnp.int32),
                  jax.ShapeDtypeStruct((N, OUT // 2), jnp.int32)),
        mesh=plsc.VectorSubcoreMesh(core_axis_name="c", subcore_axis_name="s",
                                    num_cores=NC, num_subcores=NS),
        compiler_params=pltpu.CompilerParams(use_tc_tiling_on_sc=True),
        scratch_types=[
            pltpu.VMEM((_CCH, OUT // 2), jnp.int32),
            pltpu.VMEM((_CCH, OUT // 2), jnp.int32),
            pltpu.VMEM((_CCH,), jnp.int32),
            pltpu.VMEM((_CCH,), jnp.int32),
            pltpu.SemaphoreType.DMA,
            pltpu.SemaphoreType.DMA,
        ],
    )


def _wadd_body(y1_ref, y2_ref, s1_ref, s2_ref, y_ref):
    def dec_lo(w32):
        return lax.bitcast_convert_type(w32 << 16, jnp.float32)

    def dec_hi(w32):
        return lax.bitcast_convert_type(w32 & jnp.int32(-65536), jnp.float32)

    w1 = y1_ref[...]
    w2 = y2_ref[...]
    s1 = s1_ref[:, :1]
    s2 = s2_ref[:, :1]
    y_ref[:, :OUT // 2] = s1 * dec_lo(w1) + s2 * dec_lo(w2)
    y_ref[:, OUT // 2:] = s1 * dec_hi(w1) + s2 * dec_hi(w2)


_wadd_call = pl.pallas_call(
    _wadd_body,
    grid=(NT,),
    in_specs=[
        pl.BlockSpec((T, OUT // 2), lambda i: (i, 0)),
        pl.BlockSpec((T, OUT // 2), lambda i: (i, 0)),
        pl.BlockSpec((T, 16), lambda i: (i, 0)),
        pl.BlockSpec((T, 16), lambda i: (i, 0)),
    ],
    out_specs=pl.BlockSpec((T, OUT), lambda i: (i, 0)),
    out_shape=jax.ShapeDtypeStruct((N, OUT), jnp.float32),
)


@jax.jit
def kernel(x, gate_w1, gate_w2, w_up, b_up, w_down, b_down):
    orig_shape = x.shape
    xf = x.reshape(-1, D)

    d1c, d2c, g1c, g2c, s1c, s2c, loss_arr, nblk_arr, xbf = _gate_call(
        xf, gate_w1, gate_w2)

    xd = _dispatch_call()(xbf, d1c[:, 0], d2c[:, 0])
    del nblk_arr
    yd = _ffn_call(xd, w_up, b_up.reshape(E, 1, H), w_down,
                   b_down.reshape(E, 1, OUT))
    y1p, y2p = _combine_call()(yd, g1c[:, 0], g2c[:, 0])
    y = _wadd_call(y1p, y2p, s1c, s2c)

    return y.reshape(orig_shape[:-1] + (OUT,)), loss_arr[0, 0]


# speedup vs baseline: 1.7830x; 1.1307x over previous
import functools

import jax
import jax.numpy as jnp
from jax import lax
from jax.experimental import pallas as pl
from jax.experimental.pallas import tpu as pltpu
from jax.experimental.pallas import tpu_sc as plsc

B, S, D = 2, 2048, 768
E, K, H = 64, 2, 64
OUT = 768
N = B * S
NK = N * K
C = 3 * (NK // E)
EC = E * C
T = 512
NT = N // T
FBLK = 128
FNB = C // FBLK

NC, NS = 2, 16
NW = NC * NS
TPW = N // NW


def _gate_body(x_ref, gw1_ref, gw2_ref,
               d1_ref, d2_ref, g1_ref, g2_ref, s1_ref, s2_ref, loss_ref,
               nblk_ref, xbf_ref, run_ref, imp_ref, load_ref):
    i = pl.program_id(0)

    @pl.when(i == 0)
    def _():
        run_ref[...] = jnp.zeros((1, E), jnp.float32)
        imp_ref[...] = jnp.zeros((1, E), jnp.float32)
        load_ref[...] = jnp.zeros((1, E), jnp.float32)

    xb = x_ref[...]
    xb16 = lax.bitcast_convert_type(xb.astype(jnp.bfloat16), jnp.uint16)
    lo32 = xb16[:, :D // 2].astype(jnp.uint32)
    hi32 = xb16[:, D // 2:].astype(jnp.uint32)
    xbf_ref[...] = lax.bitcast_convert_type((hi32 << 16) | lo32, jnp.int32)
    f1 = jnp.tanh(lax.dot_general(xb, gw1_ref[...], (((1,), (1,)), ((), ())),
                                  preferred_element_type=jnp.float32))
    logits = lax.dot_general(f1, gw2_ref[...], (((1,), (1,)), ((), ())),
                             preferred_element_type=jnp.float32)

    eidx = lax.broadcasted_iota(jnp.int32, (T, E), 1)
    m1 = jnp.max(logits, axis=1, keepdims=True)
    i1 = jnp.min(jnp.where(logits == m1, eidx, E), axis=1, keepdims=True)
    masked = jnp.where(eidx == i1, -jnp.inf, logits)
    m2 = jnp.max(masked, axis=1, keepdims=True)
    i2 = jnp.min(jnp.where(masked == m2, eidx, E), axis=1, keepdims=True)

    es = jnp.exp(m2 - m1)
    s1 = 1.0 / (1.0 + es)
    s2 = es * s1

    oh1 = (eidx == i1).astype(jnp.float32)
    oh2 = (eidx == i2).astype(jnp.float32)
    ohs = oh1 + oh2

    imp_ref[...] += jnp.sum(oh1 * s1 + oh2 * s2, axis=0, keepdims=True)
    load_tile = jnp.sum(ohs, axis=0, keepdims=True)
    load_ref[...] += load_tile

    rr = lax.broadcasted_iota(jnp.int32, (T, T), 0)
    cc = lax.broadcasted_iota(jnp.int32, (T, T), 1)
    tri = (cc < rr).astype(jnp.float32)
    pref = lax.dot_general(tri.astype(jnp.bfloat16), ohs.astype(jnp.bfloat16),
                           (((1,), (0,)), ((), ())),
                           preferred_element_type=jnp.float32)
    tot = run_ref[...] + pref
    rank1 = jnp.round(jnp.sum(tot * oh1, axis=1, keepdims=True))
    rank2 = jnp.round(jnp.sum(tot * oh2, axis=1, keepdims=True))
    run_ref[...] += load_tile

    cap = jnp.float32(C)
    keep1 = rank1 < cap
    keep2 = rank2 < cap
    s1_ref[...] = jnp.broadcast_to(jnp.where(keep1, s1, 0.0), (T, 16))
    s2_ref[...] = jnp.broadcast_to(jnp.where(keep2, s2, 0.0), (T, 16))
    slot1 = jnp.minimum(rank1, cap - 1.0).astype(jnp.int32)
    slot2 = jnp.minimum(rank2, cap - 1.0).astype(jnp.int32)
    comb1 = i1 * C + slot1
    comb2 = i2 * C + slot2
    g1_ref[...] = comb1.reshape(T)
    g2_ref[...] = comb2.reshape(T)
    d1_ref[...] = jnp.where(keep1, comb1, EC).reshape(T)
    d2_ref[...] = jnp.where(keep2, comb2, EC).reshape(T)

    @pl.when(i == NT - 1)
    def _():
        def cv2(v):
            mean = jnp.sum(v) / E
            var = jnp.sum((v - mean) * (v - mean)) / (E - 1)
            return var / (mean * mean + 1e-10)
        loss = (cv2(imp_ref[...]) + cv2(load_ref[...])) * 0.01
        loss_ref[...] = jnp.full((8, 128), loss, jnp.float32)
        cnt = jnp.minimum(run_ref[...], jnp.float32(C))
        nb = jnp.clip(jnp.ceil(cnt * (1.0 / FBLK)), 1.0, C // FBLK)
        pad = jnp.zeros((1, 128 - E), jnp.float32)
        nblk_ref[...] = jnp.broadcast_to(
            jnp.concatenate([nb, pad], axis=1), (8, 128)).astype(jnp.int32)


_gate_call = pl.pallas_call(
    _gate_body,
    grid=(NT,),
    in_specs=[
        pl.BlockSpec((T, D), lambda i: (i, 0)),
        pl.BlockSpec((E, D), lambda i: (0, 0)),
        pl.BlockSpec((E, E), lambda i: (0, 0)),
    ],
    out_specs=[
        pl.BlockSpec((T,), lambda i: (i,)),
        pl.BlockSpec((T,), lambda i: (i,)),
        pl.BlockSpec((T,), lambda i: (i,)),
        pl.BlockSpec((T,), lambda i: (i,)),
        pl.BlockSpec((T, 16), lambda i: (i, 0)),
        pl.BlockSpec((T, 16), lambda i: (i, 0)),
        pl.BlockSpec((8, 128), lambda i: (0, 0)),
        pl.BlockSpec((8, 128), lambda i: (0, 0)),
        pl.BlockSpec((T, D // 2), lambda i: (i, 0)),
    ],
    out_shape=[
        jax.ShapeDtypeStruct((N,), jnp.int32),
        jax.ShapeDtypeStruct((N,), jnp.int32),
        jax.ShapeDtypeStruct((N,), jnp.int32),
        jax.ShapeDtypeStruct((N,), jnp.int32),
        jax.ShapeDtypeStruct((N, 16), jnp.float32),
        jax.ShapeDtypeStruct((N, 16), jnp.float32),
        jax.ShapeDtypeStruct((8, 128), jnp.float32),
        jax.ShapeDtypeStruct((8, 128), jnp.int32),
        jax.ShapeDtypeStruct((N, D // 2), jnp.int32),
    ],
    scratch_shapes=[
        pltpu.VMEM((1, E), jnp.float32),
        pltpu.VMEM((1, E), jnp.float32),
        pltpu.VMEM((1, E), jnp.float32),
    ],
)


def _dispatch_body(x_hbm, d1_hbm, d2_hbm, xd_hbm, xbuf, d1v, d2v, sem, sem2):
    wid = lax.axis_index("s") * NC + lax.axis_index("c")
    base = wid * TPW
    pltpu.sync_copy(d1_hbm.at[pl.ds(base, TPW)], d1v)
    pltpu.sync_copy(d2_hbm.at[pl.ds(base, TPW)], d2v)
    pltpu.sync_copy(x_hbm.at[pl.ds(base, TPW)], xbuf)
    h1 = pltpu.async_copy(xbuf, xd_hbm.at[d1v], sem)
    h2 = pltpu.async_copy(xbuf, xd_hbm.at[d2v], sem2)
    h1.wait()
    h2.wait()


@functools.cache
def _dispatch_call():
    return pl.kernel(
        _dispatch_body,
        out_type=jax.ShapeDtypeStruct((EC + 8, D // 2), jnp.int32),
        mesh=plsc.VectorSubcoreMesh(core_axis_name="c", subcore_axis_name="s",
                                    num_cores=NC, num_subcores=NS),
        compiler_params=pltpu.CompilerParams(use_tc_tiling_on_sc=True),
        scratch_types=[
            pltpu.VMEM((TPW, D // 2), jnp.int32),
            pltpu.VMEM((TPW,), jnp.int32),
            pltpu.VMEM((TPW,), jnp.int32),
            pltpu.SemaphoreType.DMA,
            pltpu.SemaphoreType.DMA,
        ],
    )


def _ffn_body(xd_ref, wu_ref, bu_ref, wd_ref, bd_ref, yd_ref):
    w32 = lax.bitcast_convert_type(xd_ref[...], jnp.uint32)
    lo = lax.bitcast_convert_type((w32 & 0xFFFF).astype(jnp.uint16),
                                  jnp.bfloat16)
    hi = lax.bitcast_convert_type((w32 >> 16).astype(jnp.uint16),
                                  jnp.bfloat16)
    wu = wu_ref[0].astype(jnp.bfloat16)
    up = (lax.dot_general(lo, wu[:, :D // 2], (((1,), (1,)), ((), ())),
                          preferred_element_type=jnp.float32)
          + lax.dot_general(hi, wu[:, D // 2:], (((1,), (1,)), ((), ())),
                            preferred_element_type=jnp.float32)
          + bu_ref[0])
    h = (up * (1.0 / (1.0 + jnp.exp(-up)))).astype(jnp.bfloat16)
    dn = lax.dot_general(h, wd_ref[0].astype(jnp.bfloat16),
                         (((1,), (0,)), ((), ())),
                         preferred_element_type=jnp.float32) + bd_ref[0]
    dn16 = lax.bitcast_convert_type(dn.astype(jnp.bfloat16), jnp.uint16)
    dlo = dn16[:, :OUT // 2].astype(jnp.uint32)
    dhi = dn16[:, OUT // 2:].astype(jnp.uint32)
    yd_ref[...] = lax.bitcast_convert_type((dhi << 16) | dlo, jnp.int32)


_ffn_call = pl.pallas_call(
    _ffn_body,
    grid=(E,),
    in_specs=[
        pl.BlockSpec((C, D // 2), lambda e: (e, 0)),
        pl.BlockSpec((1, H, D), lambda e: (e, 0, 0)),
        pl.BlockSpec((1, 1, H), lambda e: (e, 0, 0)),
        pl.BlockSpec((1, H, OUT), lambda e: (e, 0, 0)),
        pl.BlockSpec((1, 1, OUT), lambda e: (e, 0, 0)),
    ],
    out_specs=pl.BlockSpec((C, OUT // 2), lambda e: (e, 0)),
    out_shape=jax.ShapeDtypeStruct((EC, OUT // 2), jnp.int32),
)


_CCH = 64


def _combine_body(yd_hbm, d1_hbm, d2_hbm, y1_hbm, y2_hbm,
                  buf1, buf2, d1v, d2v, sem1, sem2):
    wid = lax.axis_index("s") * NC + lax.axis_index("c")
    base = wid * TPW
    for ch in range(TPW // _CCH):
        off = base + ch * _CCH
        pltpu.sync_copy(d1_hbm.at[pl.ds(off, _CCH)], d1v)
        pltpu.sync_copy(d2_hbm.at[pl.ds(off, _CCH)], d2v)
        h1 = pltpu.async_copy(yd_hbm.at[d1v], buf1, sem1)
        h2 = pltpu.async_copy(yd_hbm.at[d2v], buf2, sem2)
        h1.wait()
        h2.wait()
        pltpu.sync_copy(buf1, y1_hbm.at[pl.ds(off, _CCH)])
        pltpu.sync_copy(buf2, y2_hbm.at[pl.ds(off, _CCH)])


@functools.cache
def _combine_call():
    return pl.kernel(
        _combine_body,
        out_type=(jax.ShapeDtypeStruct((N, OUT // 2), jnp.int32),
                  jax.ShapeDtypeStruct((N, OUT // 2), jnp.int32)),
        mesh=plsc.VectorSubcoreMesh(core_axis_name="c", subcore_axis_name="s",
                                    num_cores=NC, num_subcores=NS),
        compiler_params=pltpu.CompilerParams(use_tc_tiling_on_sc=True),
        scratch_types=[
            pltpu.VMEM((_CCH, OUT // 2), jnp.int32),
            pltpu.VMEM((_CCH, OUT // 2), jnp.int32),
            pltpu.VMEM((_CCH,), jnp.int32),
            pltpu.VMEM((_CCH,), jnp.int32),
            pltpu.SemaphoreType.DMA,
            pltpu.SemaphoreType.DMA,
        ],
    )


def _wadd_body(y1_ref, y2_ref, s1_ref, s2_ref, y_ref):
    def dec_lo(w32):
        return lax.bitcast_convert_type(w32 << 16, jnp.float32)

    def dec_hi(w32):
        return lax.bitcast_convert_type(w32 & jnp.int32(-65536), jnp.float32)

    w1 = y1_ref[...]
    w2 = y2_ref[...]
    s1 = s1_ref[:, :1]
    s2 = s2_ref[:, :1]
    y_ref[:, :OUT // 2] = s1 * dec_lo(w1) + s2 * dec_lo(w2)
    y_ref[:, OUT // 2:] = s1 * dec_hi(w1) + s2 * dec_hi(w2)


_wadd_call = pl.pallas_call(
    _wadd_body,
    grid=(NT,),
    in_specs=[
        pl.BlockSpec((T, OUT // 2), lambda i: (i, 0)),
        pl.BlockSpec((T, OUT // 2), lambda i: (i, 0)),
        pl.BlockSpec((T, 16), lambda i: (i, 0)),
        pl.BlockSpec((T, 16), lambda i: (i, 0)),
    ],
    out_specs=pl.BlockSpec((T, OUT), lambda i: (i, 0)),
    out_shape=jax.ShapeDtypeStruct((N, OUT), jnp.float32),
)


@jax.jit
def kernel(x, gate_w1, gate_w2, w_up, b_up, w_down, b_down):
    orig_shape = x.shape
    xf = x.reshape(-1, D)

    d1c, d2c, g1c, g2c, s1c, s2c, loss_arr, nblk_arr, xbf = _gate_call(
        xf, gate_w1, gate_w2)

    xd = _dispatch_call()(xbf, d1c, d2c)
    del nblk_arr
    yd = _ffn_call(xd, w_up, b_up.reshape(E, 1, H),
                   jnp.transpose(w_down, (0, 2, 1)),
                   b_down.reshape(E, 1, OUT))
    y1p, y2p = _combine_call()(yd, g1c, g2c)
    y = _wadd_call(y1p, y2p, s1c, s2c)

    return y.reshape(orig_shape[:-1] + (OUT,)), loss_arr[0, 0]


# speedup vs baseline: 1.7879x; 1.0028x over previous
import functools

import jax
import jax.numpy as jnp
from jax import lax
from jax.experimental import pallas as pl
from jax.experimental.pallas import tpu as pltpu
from jax.experimental.pallas import tpu_sc as plsc

B, S, D = 2, 2048, 768
E, K, H = 64, 2, 64
OUT = 768
N = B * S
NK = N * K
C = 3 * (NK // E)
EC = E * C
T = 512
NT = N // T
FBLK = 128
FNB = C // FBLK

NC, NS = 2, 16
NW = NC * NS
TPW = N // NW


def _gate_body(x_ref, gw1_ref, gw2_ref,
               d1_ref, d2_ref, g1_ref, g2_ref, s1_ref, s2_ref, loss_ref,
               nblk_ref, xbf_ref, run_ref, imp_ref, load_ref):
    i = pl.program_id(0)

    @pl.when(i == 0)
    def _():
        run_ref[...] = jnp.zeros((1, E), jnp.float32)
        imp_ref[...] = jnp.zeros((1, E), jnp.float32)
        load_ref[...] = jnp.zeros((1, E), jnp.float32)

    xb = x_ref[...]
    xb16 = lax.bitcast_convert_type(xb.astype(jnp.bfloat16), jnp.uint16)
    lo32 = xb16[:, :D // 2].astype(jnp.uint32)
    hi32 = xb16[:, D // 2:].astype(jnp.uint32)
    xbf_ref[...] = lax.bitcast_convert_type((hi32 << 16) | lo32, jnp.int32)
    f1 = jnp.tanh(lax.dot_general(xb, gw1_ref[...], (((1,), (1,)), ((), ())),
                                  preferred_element_type=jnp.float32))
    logits = lax.dot_general(f1, gw2_ref[...], (((1,), (1,)), ((), ())),
                             preferred_element_type=jnp.float32)

    eidx = lax.broadcasted_iota(jnp.int32, (T, E), 1)
    m1 = jnp.max(logits, axis=1, keepdims=True)
    i1 = jnp.min(jnp.where(logits == m1, eidx, E), axis=1, keepdims=True)
    masked = jnp.where(eidx == i1, -jnp.inf, logits)
    m2 = jnp.max(masked, axis=1, keepdims=True)
    i2 = jnp.min(jnp.where(masked == m2, eidx, E), axis=1, keepdims=True)

    es = jnp.exp(m2 - m1)
    s1 = 1.0 / (1.0 + es)
    s2 = es * s1

    oh1 = (eidx == i1).astype(jnp.float32)
    oh2 = (eidx == i2).astype(jnp.float32)
    ohs = oh1 + oh2

    imp_ref[...] += jnp.sum(oh1 * s1 + oh2 * s2, axis=0, keepdims=True)
    load_tile = jnp.sum(ohs, axis=0, keepdims=True)
    load_ref[...] += load_tile

    rr = lax.broadcasted_iota(jnp.int32, (T, T), 0)
    cc = lax.broadcasted_iota(jnp.int32, (T, T), 1)
    tri = (cc < rr).astype(jnp.float32)
    pref = lax.dot_general(tri.astype(jnp.bfloat16), ohs.astype(jnp.bfloat16),
                           (((1,), (0,)), ((), ())),
                           preferred_element_type=jnp.float32)
    tot = run_ref[...] + pref
    rank1 = jnp.round(jnp.sum(tot * oh1, axis=1, keepdims=True))
    rank2 = jnp.round(jnp.sum(tot * oh2, axis=1, keepdims=True))
    run_ref[...] += load_tile

    cap = jnp.float32(C)
    keep1 = rank1 < cap
    keep2 = rank2 < cap
    s1_ref[...] = jnp.broadcast_to(jnp.where(keep1, s1, 0.0), (T, 16))
    s2_ref[...] = jnp.broadcast_to(jnp.where(keep2, s2, 0.0), (T, 16))
    slot1 = jnp.minimum(rank1, cap - 1.0).astype(jnp.int32)
    slot2 = jnp.minimum(rank2, cap - 1.0).astype(jnp.int32)
    comb1 = i1 * C + slot1
    comb2 = i2 * C + slot2
    g1_ref[...] = comb1.reshape(T)
    g2_ref[...] = comb2.reshape(T)
    d1_ref[...] = jnp.where(keep1, comb1, EC).reshape(T)
    d2_ref[...] = jnp.where(keep2, comb2, EC).reshape(T)

    @pl.when(i == NT - 1)
    def _():
        def cv2(v):
            mean = jnp.sum(v) / E
            var = jnp.sum((v - mean) * (v - mean)) / (E - 1)
            return var / (mean * mean + 1e-10)
        loss = (cv2(imp_ref[...]) + cv2(load_ref[...])) * 0.01
        loss_ref[...] = jnp.full((8, 128), loss, jnp.float32)
        cnt = jnp.minimum(run_ref[...], jnp.float32(C))
        nb = jnp.clip(jnp.ceil(cnt * (1.0 / FBLK)), 1.0, C // FBLK)
        pad = jnp.zeros((1, 128 - E), jnp.float32)
        nblk_ref[...] = jnp.broadcast_to(
            jnp.concatenate([nb, pad], axis=1), (8, 128)).astype(jnp.int32)


_gate_call = pl.pallas_call(
    _gate_body,
    grid=(NT,),
    in_specs=[
        pl.BlockSpec((T, D), lambda i: (i, 0)),
        pl.BlockSpec((E, D), lambda i: (0, 0)),
        pl.BlockSpec((E, E), lambda i: (0, 0)),
    ],
    out_specs=[
        pl.BlockSpec((T,), lambda i: (i,)),
        pl.BlockSpec((T,), lambda i: (i,)),
        pl.BlockSpec((T,), lambda i: (i,)),
        pl.BlockSpec((T,), lambda i: (i,)),
        pl.BlockSpec((T, 16), lambda i: (i, 0)),
        pl.BlockSpec((T, 16), lambda i: (i, 0)),
        pl.BlockSpec((8, 128), lambda i: (0, 0)),
        pl.BlockSpec((8, 128), lambda i: (0, 0)),
        pl.BlockSpec((T, D // 2), lambda i: (i, 0)),
    ],
    out_shape=[
        jax.ShapeDtypeStruct((N,), jnp.int32),
        jax.ShapeDtypeStruct((N,), jnp.int32),
        jax.ShapeDtypeStruct((N,), jnp.int32),
        jax.ShapeDtypeStruct((N,), jnp.int32),
        jax.ShapeDtypeStruct((N, 16), jnp.float32),
        jax.ShapeDtypeStruct((N, 16), jnp.float32),
        jax.ShapeDtypeStruct((8, 128), jnp.float32),
        jax.ShapeDtypeStruct((8, 128), jnp.int32),
        jax.ShapeDtypeStruct((N, D // 2), jnp.int32),
    ],
    scratch_shapes=[
        pltpu.VMEM((1, E), jnp.float32),
        pltpu.VMEM((1, E), jnp.float32),
        pltpu.VMEM((1, E), jnp.float32),
    ],
)


def _dispatch_body(x_hbm, d1_hbm, d2_hbm, xd_hbm, xbuf, d1v, d2v, sem, sem2):
    wid = lax.axis_index("s") * NC + lax.axis_index("c")
    base = wid * TPW
    pltpu.sync_copy(d1_hbm.at[pl.ds(base, TPW)], d1v)
    pltpu.sync_copy(d2_hbm.at[pl.ds(base, TPW)], d2v)
    pltpu.sync_copy(x_hbm.at[pl.ds(base, TPW)], xbuf)
    h1 = pltpu.async_copy(xbuf, xd_hbm.at[d1v], sem)
    h2 = pltpu.async_copy(xbuf, xd_hbm.at[d2v], sem2)
    h1.wait()
    h2.wait()


@functools.cache
def _dispatch_call():
    return pl.kernel(
        _dispatch_body,
        out_type=jax.ShapeDtypeStruct((EC + 8, D // 2), jnp.int32),
        mesh=plsc.VectorSubcoreMesh(core_axis_name="c", subcore_axis_name="s",
                                    num_cores=NC, num_subcores=NS),
        scratch_types=[
            pltpu.VMEM((TPW, D // 2), jnp.int32),
            pltpu.VMEM((TPW,), jnp.int32),
            pltpu.VMEM((TPW,), jnp.int32),
            pltpu.SemaphoreType.DMA,
            pltpu.SemaphoreType.DMA,
        ],
    )


def _ffn_body(xd_ref, wu_ref, bu_ref, wd_ref, bd_ref, yd_ref):
    w32 = lax.bitcast_convert_type(xd_ref[...], jnp.uint32)
    lo = lax.bitcast_convert_type((w32 & 0xFFFF).astype(jnp.uint16),
                                  jnp.bfloat16)
    hi = lax.bitcast_convert_type((w32 >> 16).astype(jnp.uint16),
                                  jnp.bfloat16)
    wu = wu_ref[0].astype(jnp.bfloat16)
    up = (lax.dot_general(lo, wu[:, :D // 2], (((1,), (1,)), ((), ())),
                          preferred_element_type=jnp.float32)
          + lax.dot_general(hi, wu[:, D // 2:], (((1,), (1,)), ((), ())),
                            preferred_element_type=jnp.float32)
          + bu_ref[0])
    h = (up * (1.0 / (1.0 + jnp.exp(-up)))).astype(jnp.bfloat16)
    dn = lax.dot_general(h, wd_ref[0].astype(jnp.bfloat16),
                         (((1,), (0,)), ((), ())),
                         preferred_element_type=jnp.float32) + bd_ref[0]
    dn16 = lax.bitcast_convert_type(dn.astype(jnp.bfloat16), jnp.uint16)
    dlo = dn16[:, :OUT // 2].astype(jnp.uint32)
    dhi = dn16[:, OUT // 2:].astype(jnp.uint32)
    yd_ref[...] = lax.bitcast_convert_type((dhi << 16) | dlo, jnp.int32)


_ffn_call = pl.pallas_call(
    _ffn_body,
    grid=(E,),
    in_specs=[
        pl.BlockSpec((C, D // 2), lambda e: (e, 0)),
        pl.BlockSpec((1, H, D), lambda e: (e, 0, 0)),
        pl.BlockSpec((1, 1, H), lambda e: (e, 0, 0)),
        pl.BlockSpec((1, H, OUT), lambda e: (e, 0, 0)),
        pl.BlockSpec((1, 1, OUT), lambda e: (e, 0, 0)),
    ],
    out_specs=pl.BlockSpec((C, OUT // 2), lambda e: (e, 0)),
    out_shape=jax.ShapeDtypeStruct((EC, OUT // 2), jnp.int32),
)


_CCH = 64


def _combine_body(yd_hbm, d1_hbm, d2_hbm, y1_hbm, y2_hbm,
                  buf1, buf2, d1v, d2v, sem1, sem2):
    wid = lax.axis_index("s") * NC + lax.axis_index("c")
    base = wid * TPW
    for ch in range(TPW // _CCH):
        off = base + ch * _CCH
        pltpu.sync_copy(d1_hbm.at[pl.ds(off, _CCH)], d1v)
        pltpu.sync_copy(d2_hbm.at[pl.ds(off, _CCH)], d2v)
        h1 = pltpu.async_copy(yd_hbm.at[d1v], buf1, sem1)
        h2 = pltpu.async_copy(yd_hbm.at[d2v], buf2, sem2)
        h1.wait()
        h2.wait()
        pltpu.sync_copy(buf1, y1_hbm.at[pl.ds(off, _CCH)])
        pltpu.sync_copy(buf2, y2_hbm.at[pl.ds(off, _CCH)])


@functools.cache
def _combine_call():
    return pl.kernel(
        _combine_body,
        out_type=(jax.ShapeDtypeStruct((N, OUT // 2), jnp.int32),
                  jax.ShapeDtypeStruct((N, OUT // 2), jnp.int32)),
        mesh=plsc.VectorSubcoreMesh(core_axis_name="c", subcore_axis_name="s",
                                    num_cores=NC, num_subcores=NS),
        scratch_types=[
            pltpu.VMEM((_CCH, OUT // 2), jnp.int32),
            pltpu.VMEM((_CCH, OUT // 2), jnp.int32),
            pltpu.VMEM((_CCH,), jnp.int32),
            pltpu.VMEM((_CCH,), jnp.int32),
            pltpu.SemaphoreType.DMA,
            pltpu.SemaphoreType.DMA,
        ],
    )


def _wadd_body(y1_ref, y2_ref, s1_ref, s2_ref, y_ref):
    def dec_lo(w32):
        return lax.bitcast_convert_type(w32 << 16, jnp.float32)

    def dec_hi(w32):
        return lax.bitcast_convert_type(w32 & jnp.int32(-65536), jnp.float32)

    w1 = y1_ref[...]
    w2 = y2_ref[...]
    s1 = s1_ref[:, :1]
    s2 = s2_ref[:, :1]
    y_ref[:, :OUT // 2] = s1 * dec_lo(w1) + s2 * dec_lo(w2)
    y_ref[:, OUT // 2:] = s1 * dec_hi(w1) + s2 * dec_hi(w2)


_wadd_call = pl.pallas_call(
    _wadd_body,
    grid=(NT,),
    in_specs=[
        pl.BlockSpec((T, OUT // 2), lambda i: (i, 0)),
        pl.BlockSpec((T, OUT // 2), lambda i: (i, 0)),
        pl.BlockSpec((T, 16), lambda i: (i, 0)),
        pl.BlockSpec((T, 16), lambda i: (i, 0)),
    ],
    out_specs=pl.BlockSpec((T, OUT), lambda i: (i, 0)),
    out_shape=jax.ShapeDtypeStruct((N, OUT), jnp.float32),
)


@jax.jit
def kernel(x, gate_w1, gate_w2, w_up, b_up, w_down, b_down):
    orig_shape = x.shape
    xf = x.reshape(-1, D)

    d1c, d2c, g1c, g2c, s1c, s2c, loss_arr, nblk_arr, xbf = _gate_call(
        xf, gate_w1, gate_w2)

    xd = _dispatch_call()(xbf, d1c, d2c)
    del nblk_arr
    yd = _ffn_call(xd, w_up, b_up.reshape(E, 1, H),
                   jnp.transpose(w_down, (0, 2, 1)),
                   b_down.reshape(E, 1, OUT))
    y1p, y2p = _combine_call()(yd, g1c, g2c)
    y = _wadd_call(y1p, y2p, s1c, s2c)

    return y.reshape(orig_shape[:-1] + (OUT,)), loss_arr[0, 0]
